# baseline (speedup 1.0000x reference)
import functools

import jax
import jax.numpy as jnp
from jax import lax
from jax.experimental import pallas as pl
from jax.experimental.pallas import tpu as pltpu
from jax.experimental.pallas import tpu_sc as plsc

N_NODES = 10000
NFEAT = 128
NHID = 128

NC = 2
NS = 16
NW = NC * NS
L = 16

N_PAD = 10240
ROWS_PER_TILE = N_PAD // NS

N_EDGES = 320000
CHUNK = 128
NCHUNK_TOT = N_EDGES // CHUNK
NCHUNK = 80
H = NCHUNK // 2
VPC = CHUNK // L
PART = NCHUNK_TOT - (NW - 1) * NCHUNK

_mesh = plsc.VectorSubcoreMesh(core_axis_name="c", subcore_axis_name="s")


def _nh(wid, h):
    n_real = jnp.clip(NCHUNK_TOT - wid * NCHUNK, 0, NCHUNK)
    return jnp.clip(n_real - h * H, 0, H)


@functools.partial(
    pl.kernel,
    out_type=(
        jax.ShapeDtypeStruct((NW, N_PAD), jnp.float32),
        jax.ShapeDtypeStruct((NW * NCHUNK, CHUNK), jnp.int32),
    ),
    mesh=_mesh,
    scratch_types=[
        pltpu.VMEM((H * CHUNK,), jnp.int32),
        pltpu.VMEM((H, CHUNK), jnp.int32),
        pltpu.VMEM((N_PAD,), jnp.float32),
    ],
    compiler_params=pltpu.CompilerParams(needs_layout_passes=False),
)
def _deg_kernel(ei_hbm, out_hbm, dst2_hbm, idxf_v, idx2_v, deg_v):
    c = lax.axis_index("c")
    s = lax.axis_index("s")
    wid = c * NS + s
    zeros = jnp.zeros((L,), jnp.float32)
    ones = jnp.ones((L,), jnp.float32)

    def zero_body(i, _):
        for u in range(4):
            deg_v[pl.ds((4 * i + u) * L, L)] = zeros
        return 0

    lax.fori_loop(0, N_PAD // L // 4, zero_body, 0)

    for h in range(2):
        nh = _nh(wid, h)
        e0 = wid * NCHUNK * CHUNK + h * H * CHUNK

        @pl.when(nh >= H)
        def _full():
            pltpu.sync_copy(ei_hbm.at[1, pl.ds(e0, H * CHUNK)], idxf_v)

        @pl.when(jnp.logical_and(nh > 0, nh < H))
        def _tail():
            pltpu.sync_copy(ei_hbm.at[1, pl.ds(e0, PART * CHUNK)],
                            idxf_v.at[pl.ds(0, PART * CHUNK)])

        @pl.when(nh > 0)
        def _half():
            def body(i4, _):
                for u in range(4):
                    i = i4 * 4 + u
                    idx = idxf_v[pl.ds(i * L, L)]
                    plsc.addupdate_scatter(deg_v, [idx], ones)
                    idx2_v[i // VPC, pl.ds((i % VPC) * L, L)] = idx
                return 0

            lax.fori_loop(0, nh * VPC // 4, body, 0)
            pltpu.sync_copy(idx2_v,
                            dst2_hbm.at[pl.ds(wid * NCHUNK + h * H, H)])

    pltpu.sync_copy(deg_v, out_hbm.at[wid])


def _mm_body(x_ref, w_ref, h_ref):
    h_ref[...] = jnp.dot(x_ref[...], w_ref[...],
                         preferred_element_type=jnp.float32)


def _mm(x, w):
    return pl.pallas_call(
        _mm_body,
        out_shape=jax.ShapeDtypeStruct((N_NODES, NHID), jnp.float32),
    )(x, w)


def _dinv_body(degp_ref, dinv_ref):
    deg = jnp.sum(degp_ref[...], axis=0) + 1.0
    dinv_ref[...] = lax.rsqrt(deg)[:, None]


def _dinv(degp):
    return pl.pallas_call(
        _dinv_body,
        out_shape=jax.ShapeDtypeStruct((N_PAD, 1), jnp.float32),
    )(degp)


def _gscale_body(h_ref, dinv_ref, g_ref):
    g_ref[...] = h_ref[...] * dinv_ref[...]


def _gscale(h, dinv):
    blk = 2000
    return pl.pallas_call(
        _gscale_body,
        grid=(N_NODES // blk,),
        in_specs=[
            pl.BlockSpec((blk, NHID), lambda i: (i, 0)),
            pl.BlockSpec((blk, 1), lambda i: (i, 0)),
        ],
        out_specs=pl.BlockSpec((blk, NHID), lambda i: (i, 0)),
        out_shape=jax.ShapeDtypeStruct((N_NODES, NHID), jnp.float32),
    )(h, dinv)


@functools.partial(
    pl.kernel,
    out_type=jax.ShapeDtypeStruct((NC, N_PAD, NHID), jnp.float32),
    mesh=_mesh,
    scratch_types=[
        pltpu.VMEM((H * CHUNK,), jnp.int32),
        pltpu.VMEM((H, CHUNK), jnp.int32),
        pltpu.VMEM((2, CHUNK, NHID), jnp.float32),
        pltpu.VMEM_SHARED((N_PAD, NHID), jnp.float32),
        pltpu.SemaphoreType.DMA,
        pltpu.SemaphoreType.DMA,
    ],
)
def _edge_kernel(ei_hbm, dst2_hbm, g_hbm, zeros_hbm, out_hbm,
                 src_v, dst_v, rows_v, acc_sh, sem, sem_s):
    c = lax.axis_index("c")
    s = lax.axis_index("s")
    wid = c * NS + s

    pltpu.sync_copy(zeros_hbm, rows_v.at[0])
    base_row = s * ROWS_PER_TILE
    NZ = ROWS_PER_TILE // CHUNK
    for k in range(NZ):
        pltpu.async_copy(rows_v.at[0],
                         acc_sh.at[pl.ds(base_row + k * CHUNK, CHUNK)],
                         sem_s)

    nh0 = _nh(wid, 0)
    e00 = wid * NCHUNK * CHUNK

    @pl.when(nh0 >= H)
    def _full0():
        pltpu.sync_copy(ei_hbm.at[0, pl.ds(e00, H * CHUNK)], src_v)

    @pl.when(jnp.logical_and(nh0 > 0, nh0 < H))
    def _tail0():
        pltpu.sync_copy(ei_hbm.at[0, pl.ds(e00, PART * CHUNK)],
                        src_v.at[pl.ds(0, PART * CHUNK)])

    @pl.when(nh0 > 0)
    def _dst0():
        pltpu.sync_copy(dst2_hbm.at[pl.ds(wid * NCHUNK, H)], dst_v)

    for k in range(NZ):
        pltpu.make_async_copy(
            rows_v.at[0], acc_sh.at[pl.ds(base_row + k * CHUNK, CHUNK)],
            sem_s).wait()
    plsc.subcore_barrier()

    for h in range(2):
        nh = _nh(wid, h)
        e0 = wid * NCHUNK * CHUNK + h * H * CHUNK

        if h == 1:
            @pl.when(nh >= H)
            def _full():
                pltpu.sync_copy(ei_hbm.at[0, pl.ds(e0, H * CHUNK)], src_v)

            @pl.when(jnp.logical_and(nh > 0, nh < H))
            def _tail():
                pltpu.sync_copy(ei_hbm.at[0, pl.ds(e0, PART * CHUNK)],
                                src_v.at[pl.ds(0, PART * CHUNK)])

            @pl.when(nh > 0)
            def _dsth():
                pltpu.sync_copy(
                    dst2_hbm.at[pl.ds(wid * NCHUNK + h * H, H)], dst_v)

        @pl.when(nh > 0)
        def _half():
            pltpu.async_copy(g_hbm.at[src_v.at[pl.ds(0, CHUNK)]],
                             rows_v.at[0], sem)

            def chunk_body(j, _):
                nxt = j + 1
                pltpu.async_copy(
                    g_hbm.at[src_v.at[pl.ds(nxt * CHUNK, CHUNK)]],
                    rows_v.at[nxt & 1], sem)
                pltpu.make_async_copy(
                    g_hbm.at[src_v.at[pl.ds(j * CHUNK, CHUNK)]],
                    rows_v.at[j & 1], sem).wait()
                pltpu.sync_copy(rows_v.at[j & 1], acc_sh.at[dst_v.at[j]],
                                add=True)
                return 0

            lax.fori_loop(0, nh - 1, chunk_body, 0)
            last = nh - 1
            pltpu.make_async_copy(
                g_hbm.at[src_v.at[pl.ds(last * CHUNK, CHUNK)]],
                rows_v.at[last & 1], sem).wait()
            pltpu.sync_copy(rows_v.at[last & 1], acc_sh.at[dst_v.at[last]],
                            add=True)

    plsc.subcore_barrier()
    pltpu.sync_copy(acc_sh.at[pl.ds(base_row, ROWS_PER_TILE)],
                    out_hbm.at[c, pl.ds(base_row, ROWS_PER_TILE)])


def _tc2_body(accp_ref, g_ref, dinv_ref, b_ref, out_ref):
    ssum = accp_ref[0] + accp_ref[1] + g_ref[...]
    out_ref[...] = ssum * dinv_ref[...] + b_ref[...]


def _tc2(accp, g, dinv, b):
    blk = 2000
    return pl.pallas_call(
        _tc2_body,
        grid=(N_NODES // blk,),
        in_specs=[
            pl.BlockSpec((2, blk, NHID), lambda i: (0, i, 0)),
            pl.BlockSpec((blk, NHID), lambda i: (i, 0)),
            pl.BlockSpec((blk, 1), lambda i: (i, 0)),
            pl.BlockSpec((1, NHID), lambda i: (0, 0)),
        ],
        out_specs=pl.BlockSpec((blk, NHID), lambda i: (i, 0)),
        out_shape=jax.ShapeDtypeStruct((N_NODES, NHID), jnp.float32),
    )(accp, g, dinv, b)


@jax.jit
def kernel(x, edge_index, W, b):
    ei = edge_index.astype(jnp.int32)
    degp, dst2 = _deg_kernel(ei)
    hmat = _mm(x, W)
    dinv = _dinv(degp)
    g = _gscale(hmat, dinv)
    zc = jnp.zeros((CHUNK, NHID), jnp.float32)
    accp = _edge_kernel(ei, dst2, g, zc)
    return _tc2(accp, g, dinv, b.reshape(1, NHID))


# speedup vs baseline: 1.0329x; 1.0329x over previous
import functools

import jax
import jax.numpy as jnp
from jax import lax
from jax.experimental import pallas as pl
from jax.experimental.pallas import tpu as pltpu
from jax.experimental.pallas import tpu_sc as plsc

N_NODES = 10000
NFEAT = 128
NHID = 128

NC = 2
NS = 16
NW = NC * NS
L = 16

N_PAD = 10240
ROWS_PER_TILE = N_PAD // NS

N_EDGES = 320000
CHUNK = 128
NCHUNK_TOT = N_EDGES // CHUNK
NCHUNK = 80
H = NCHUNK // 2
VPC = CHUNK // L
PART = NCHUNK_TOT - (NW - 1) * NCHUNK

_mesh = plsc.VectorSubcoreMesh(core_axis_name="c", subcore_axis_name="s")


def _nh(wid, h):
    n_real = jnp.clip(NCHUNK_TOT - wid * NCHUNK, 0, NCHUNK)
    return jnp.clip(n_real - h * H, 0, H)


@functools.partial(
    pl.kernel,
    out_type=(
        jax.ShapeDtypeStruct((NW, N_PAD), jnp.float32),
        jax.ShapeDtypeStruct((NW * NCHUNK, CHUNK), jnp.int32),
    ),
    mesh=_mesh,
    scratch_types=[
        pltpu.VMEM((H * CHUNK,), jnp.int32),
        pltpu.VMEM((H, CHUNK), jnp.int32),
        pltpu.VMEM((N_PAD,), jnp.float32),
    ],
    compiler_params=pltpu.CompilerParams(needs_layout_passes=False),
)
def _deg_kernel(ei_hbm, out_hbm, dst2_hbm, idxf_v, idx2_v, deg_v):
    c = lax.axis_index("c")
    s = lax.axis_index("s")
    wid = c * NS + s
    zeros = jnp.zeros((L,), jnp.float32)
    ones = jnp.ones((L,), jnp.float32)

    def zero_body(i, _):
        for u in range(4):
            deg_v[pl.ds((4 * i + u) * L, L)] = zeros
        return 0

    lax.fori_loop(0, N_PAD // L // 4, zero_body, 0)

    for h in range(2):
        nh = _nh(wid, h)
        e0 = wid * NCHUNK * CHUNK + h * H * CHUNK

        @pl.when(nh >= H)
        def _full():
            pltpu.sync_copy(ei_hbm.at[1, pl.ds(e0, H * CHUNK)], idxf_v)

        @pl.when(jnp.logical_and(nh > 0, nh < H))
        def _tail():
            pltpu.sync_copy(ei_hbm.at[1, pl.ds(e0, PART * CHUNK)],
                            idxf_v.at[pl.ds(0, PART * CHUNK)])

        @pl.when(nh > 0)
        def _half():
            def body(i4, _):
                for u in range(4):
                    i = i4 * 4 + u
                    idx = idxf_v[pl.ds(i * L, L)]
                    plsc.addupdate_scatter(deg_v, [idx], ones)
                    idx2_v[i // VPC, pl.ds((i % VPC) * L, L)] = idx
                return 0

            lax.fori_loop(0, nh * VPC // 4, body, 0)
            pltpu.sync_copy(idx2_v,
                            dst2_hbm.at[pl.ds(wid * NCHUNK + h * H, H)])

    pltpu.sync_copy(deg_v, out_hbm.at[wid])


def _mm_body(x_ref, w_ref, h_ref):
    h_ref[...] = jnp.dot(x_ref[...], w_ref[...],
                         preferred_element_type=jnp.float32)


def _mm(x, w):
    return pl.pallas_call(
        _mm_body,
        out_shape=jax.ShapeDtypeStruct((N_NODES, NHID), jnp.float32),
    )(x, w)


def _scale_body(degp_ref, h_ref, g_ref, dinv_ref):
    deg = jnp.sum(degp_ref[...], axis=0) + 1.0
    dinv = lax.rsqrt(deg)[:N_NODES]
    g_ref[...] = h_ref[...] * dinv[:, None]
    dinv_ref[...] = dinv[:, None]


def _scale(degp, h):
    return pl.pallas_call(
        _scale_body,
        out_shape=(
            jax.ShapeDtypeStruct((N_NODES, NHID), jnp.float32),
            jax.ShapeDtypeStruct((N_NODES, 1), jnp.float32),
        ),
    )(degp, h)


@functools.partial(
    pl.kernel,
    out_type=jax.ShapeDtypeStruct((NC, N_PAD, NHID), jnp.float32),
    mesh=_mesh,
    scratch_types=[
        pltpu.VMEM((H * CHUNK,), jnp.int32),
        pltpu.VMEM((H, CHUNK), jnp.int32),
        pltpu.VMEM((2, CHUNK, NHID), jnp.float32),
        pltpu.VMEM_SHARED((N_PAD, NHID), jnp.float32),
        pltpu.SemaphoreType.DMA,
        pltpu.SemaphoreType.DMA,
    ],
)
def _edge_kernel(ei_hbm, dst2_hbm, g_hbm, zeros_hbm, out_hbm,
                 src_v, dst_v, rows_v, acc_sh, sem, sem_s):
    c = lax.axis_index("c")
    s = lax.axis_index("s")
    wid = c * NS + s

    pltpu.sync_copy(zeros_hbm, rows_v.at[0])
    base_row = s * ROWS_PER_TILE
    NZ = ROWS_PER_TILE // CHUNK
    for k in range(NZ):
        pltpu.async_copy(rows_v.at[0],
                         acc_sh.at[pl.ds(base_row + k * CHUNK, CHUNK)],
                         sem_s)

    nh0 = _nh(wid, 0)
    e00 = wid * NCHUNK * CHUNK

    @pl.when(nh0 >= H)
    def _full0():
        pltpu.sync_copy(ei_hbm.at[0, pl.ds(e00, H * CHUNK)], src_v)

    @pl.when(jnp.logical_and(nh0 > 0, nh0 < H))
    def _tail0():
        pltpu.sync_copy(ei_hbm.at[0, pl.ds(e00, PART * CHUNK)],
                        src_v.at[pl.ds(0, PART * CHUNK)])

    @pl.when(nh0 > 0)
    def _dst0():
        pltpu.sync_copy(dst2_hbm.at[pl.ds(wid * NCHUNK, H)], dst_v)

    for k in range(NZ):
        pltpu.make_async_copy(
            rows_v.at[0], acc_sh.at[pl.ds(base_row + k * CHUNK, CHUNK)],
            sem_s).wait()
    plsc.subcore_barrier()

    for h in range(2):
        nh = _nh(wid, h)
        e0 = wid * NCHUNK * CHUNK + h * H * CHUNK

        if h == 1:
            @pl.when(nh >= H)
            def _full():
                pltpu.sync_copy(ei_hbm.at[0, pl.ds(e0, H * CHUNK)], src_v)

            @pl.when(jnp.logical_and(nh > 0, nh < H))
            def _tail():
                pltpu.sync_copy(ei_hbm.at[0, pl.ds(e0, PART * CHUNK)],
                                src_v.at[pl.ds(0, PART * CHUNK)])

            @pl.when(nh > 0)
            def _dsth():
                pltpu.sync_copy(
                    dst2_hbm.at[pl.ds(wid * NCHUNK + h * H, H)], dst_v)

        @pl.when(nh > 0)
        def _half():
            pltpu.async_copy(g_hbm.at[src_v.at[pl.ds(0, CHUNK)]],
                             rows_v.at[0], sem)

            def chunk_body(j, _):
                nxt = j + 1
                pltpu.async_copy(
                    g_hbm.at[src_v.at[pl.ds(nxt * CHUNK, CHUNK)]],
                    rows_v.at[nxt & 1], sem)
                pltpu.make_async_copy(
                    g_hbm.at[src_v.at[pl.ds(j * CHUNK, CHUNK)]],
                    rows_v.at[j & 1], sem).wait()
                pltpu.sync_copy(rows_v.at[j & 1], acc_sh.at[dst_v.at[j]],
                                add=True)
                return 0

            lax.fori_loop(0, nh - 1, chunk_body, 0)
            last = nh - 1
            pltpu.make_async_copy(
                g_hbm.at[src_v.at[pl.ds(last * CHUNK, CHUNK)]],
                rows_v.at[last & 1], sem).wait()
            pltpu.sync_copy(rows_v.at[last & 1], acc_sh.at[dst_v.at[last]],
                            add=True)

    plsc.subcore_barrier()
    pltpu.sync_copy(acc_sh.at[pl.ds(base_row, ROWS_PER_TILE)],
                    out_hbm.at[c, pl.ds(base_row, ROWS_PER_TILE)])


def _tc2_body(accp_ref, g_ref, dinv_ref, b_ref, out_ref):
    ssum = accp_ref[0] + accp_ref[1] + g_ref[...]
    out_ref[...] = ssum * dinv_ref[...] + b_ref[...]


def _tc2(accp, g, dinv, b):
    blk = 2000
    return pl.pallas_call(
        _tc2_body,
        grid=(N_NODES // blk,),
        in_specs=[
            pl.BlockSpec((2, blk, NHID), lambda i: (0, i, 0)),
            pl.BlockSpec((blk, NHID), lambda i: (i, 0)),
            pl.BlockSpec((blk, 1), lambda i: (i, 0)),
            pl.BlockSpec((1, NHID), lambda i: (0, 0)),
        ],
        out_specs=pl.BlockSpec((blk, NHID), lambda i: (i, 0)),
        out_shape=jax.ShapeDtypeStruct((N_NODES, NHID), jnp.float32),
    )(accp, g, dinv, b)


@jax.jit
def kernel(x, edge_index, W, b):
    ei = edge_index.astype(jnp.int32)
    degp, dst2 = _deg_kernel(ei)
    hmat = _mm(x, W)
    g, dinv = _scale(degp, hmat)
    zc = jnp.zeros((CHUNK, NHID), jnp.float32)
    accp = _edge_kernel(ei, dst2, g, zc)
    return _tc2(accp, g, dinv, b.reshape(1, NHID))


# speedup vs baseline: 1.0373x; 1.0043x over previous
import functools

import jax
import jax.numpy as jnp
from jax import lax
from jax.experimental import pallas as pl
from jax.experimental.pallas import tpu as pltpu
from jax.experimental.pallas import tpu_sc as plsc

N_NODES = 10000
NFEAT = 128
NHID = 128

NC = 2
NS = 16
NW = NC * NS
L = 16

N_PAD = 10240
ROWS_PER_TILE = N_PAD // NS

N_EDGES = 320000
CHUNK = 128
NCHUNK_TOT = N_EDGES // CHUNK
NCHUNK = 80
H = NCHUNK // 2
VPC = CHUNK // L
PART = NCHUNK_TOT - (NW - 1) * NCHUNK

_mesh = plsc.VectorSubcoreMesh(core_axis_name="c", subcore_axis_name="s")


def _nh(wid, h):
    n_real = jnp.clip(NCHUNK_TOT - wid * NCHUNK, 0, NCHUNK)
    return jnp.clip(n_real - h * H, 0, H)


@functools.partial(
    pl.kernel,
    out_type=(
        jax.ShapeDtypeStruct((NW, N_PAD), jnp.float32),
        jax.ShapeDtypeStruct((NW * NCHUNK, CHUNK), jnp.int32),
    ),
    mesh=_mesh,
    scratch_types=[
        pltpu.VMEM((H * CHUNK,), jnp.int32),
        pltpu.VMEM((H, CHUNK), jnp.int32),
        pltpu.VMEM((N_PAD,), jnp.float32),
    ],
    compiler_params=pltpu.CompilerParams(needs_layout_passes=False),
)
def _deg_kernel(ei_hbm, out_hbm, dst2_hbm, idxf_v, idx2_v, deg_v):
    c = lax.axis_index("c")
    s = lax.axis_index("s")
    wid = c * NS + s
    zeros = jnp.zeros((L,), jnp.float32)
    ones = jnp.ones((L,), jnp.float32)

    def zero_body(i, _):
        for u in range(4):
            deg_v[pl.ds((4 * i + u) * L, L)] = zeros
        return 0

    lax.fori_loop(0, N_PAD // L // 4, zero_body, 0)

    for h in range(2):
        nh = _nh(wid, h)
        e0 = wid * NCHUNK * CHUNK + h * H * CHUNK

        @pl.when(nh >= H)
        def _full():
            pltpu.sync_copy(ei_hbm.at[1, pl.ds(e0, H * CHUNK)], idxf_v)

        @pl.when(jnp.logical_and(nh > 0, nh < H))
        def _tail():
            pltpu.sync_copy(ei_hbm.at[1, pl.ds(e0, PART * CHUNK)],
                            idxf_v.at[pl.ds(0, PART * CHUNK)])

        @pl.when(nh > 0)
        def _half():
            def body(i4, _):
                for u in range(4):
                    i = i4 * 4 + u
                    idx = idxf_v[pl.ds(i * L, L)]
                    plsc.addupdate_scatter(deg_v, [idx], ones)
                    idx2_v[i // VPC, pl.ds((i % VPC) * L, L)] = idx
                return 0

            lax.fori_loop(0, nh * VPC // 4, body, 0)
            pltpu.sync_copy(idx2_v,
                            dst2_hbm.at[pl.ds(wid * NCHUNK + h * H, H)])

    pltpu.sync_copy(deg_v, out_hbm.at[wid])


def _scale_body(degp_ref, x_ref, w_ref, g_ref, dinv_ref):
    deg = jnp.sum(degp_ref[...], axis=0) + 1.0
    dinv = lax.rsqrt(deg)[:N_NODES]
    h = jnp.dot(x_ref[...], w_ref[...], preferred_element_type=jnp.float32)
    g_ref[...] = h * dinv[:, None]
    dinv_ref[...] = dinv[:, None]


def _scale(degp, x, w):
    return pl.pallas_call(
        _scale_body,
        out_shape=(
            jax.ShapeDtypeStruct((N_NODES, NHID), jnp.float32),
            jax.ShapeDtypeStruct((N_NODES, 1), jnp.float32),
        ),
    )(degp, x, w)


@functools.partial(
    pl.kernel,
    out_type=jax.ShapeDtypeStruct((NC, N_PAD, NHID), jnp.float32),
    mesh=_mesh,
    scratch_types=[
        pltpu.VMEM((H * CHUNK,), jnp.int32),
        pltpu.VMEM((H, CHUNK), jnp.int32),
        pltpu.VMEM((2, CHUNK, NHID), jnp.float32),
        pltpu.VMEM_SHARED((N_PAD, NHID), jnp.float32),
        pltpu.SemaphoreType.DMA,
        pltpu.SemaphoreType.DMA,
    ],
)
def _edge_kernel(ei_hbm, dst2_hbm, g_hbm, zeros_hbm, out_hbm,
                 src_v, dst_v, rows_v, acc_sh, sem, sem_s):
    c = lax.axis_index("c")
    s = lax.axis_index("s")
    wid = c * NS + s

    pltpu.sync_copy(zeros_hbm, rows_v.at[0])
    base_row = s * ROWS_PER_TILE
    NZ = ROWS_PER_TILE // CHUNK
    for k in range(NZ):
        pltpu.async_copy(rows_v.at[0],
                         acc_sh.at[pl.ds(base_row + k * CHUNK, CHUNK)],
                         sem_s)

    nh0 = _nh(wid, 0)
    e00 = wid * NCHUNK * CHUNK

    @pl.when(nh0 >= H)
    def _full0():
        pltpu.sync_copy(ei_hbm.at[0, pl.ds(e00, H * CHUNK)], src_v)

    @pl.when(jnp.logical_and(nh0 > 0, nh0 < H))
    def _tail0():
        pltpu.sync_copy(ei_hbm.at[0, pl.ds(e00, PART * CHUNK)],
                        src_v.at[pl.ds(0, PART * CHUNK)])

    @pl.when(nh0 > 0)
    def _dst0():
        pltpu.sync_copy(dst2_hbm.at[pl.ds(wid * NCHUNK, H)], dst_v)

    for k in range(NZ):
        pltpu.make_async_copy(
            rows_v.at[0], acc_sh.at[pl.ds(base_row + k * CHUNK, CHUNK)],
            sem_s).wait()
    plsc.subcore_barrier()

    for h in range(2):
        nh = _nh(wid, h)
        e0 = wid * NCHUNK * CHUNK + h * H * CHUNK

        if h == 1:
            @pl.when(nh >= H)
            def _full():
                pltpu.sync_copy(ei_hbm.at[0, pl.ds(e0, H * CHUNK)], src_v)

            @pl.when(jnp.logical_and(nh > 0, nh < H))
            def _tail():
                pltpu.sync_copy(ei_hbm.at[0, pl.ds(e0, PART * CHUNK)],
                                src_v.at[pl.ds(0, PART * CHUNK)])

            @pl.when(nh > 0)
            def _dsth():
                pltpu.sync_copy(
                    dst2_hbm.at[pl.ds(wid * NCHUNK + h * H, H)], dst_v)

        @pl.when(nh > 0)
        def _half():
            pltpu.async_copy(g_hbm.at[src_v.at[pl.ds(0, CHUNK)]],
                             rows_v.at[0], sem)

            def chunk_body(j, _):
                nxt = j + 1
                pltpu.async_copy(
                    g_hbm.at[src_v.at[pl.ds(nxt * CHUNK, CHUNK)]],
                    rows_v.at[nxt & 1], sem)
                pltpu.make_async_copy(
                    g_hbm.at[src_v.at[pl.ds(j * CHUNK, CHUNK)]],
                    rows_v.at[j & 1], sem).wait()
                pltpu.sync_copy(rows_v.at[j & 1], acc_sh.at[dst_v.at[j]],
                                add=True)
                return 0

            lax.fori_loop(0, nh - 1, chunk_body, 0)
            last = nh - 1
            pltpu.make_async_copy(
                g_hbm.at[src_v.at[pl.ds(last * CHUNK, CHUNK)]],
                rows_v.at[last & 1], sem).wait()
            pltpu.sync_copy(rows_v.at[last & 1], acc_sh.at[dst_v.at[last]],
                            add=True)

    plsc.subcore_barrier()
    pltpu.sync_copy(acc_sh.at[pl.ds(base_row, ROWS_PER_TILE)],
                    out_hbm.at[c, pl.ds(base_row, ROWS_PER_TILE)])


def _tc2_body(accp_ref, g_ref, dinv_ref, b_ref, out_ref):
    ssum = accp_ref[0] + accp_ref[1] + g_ref[...]
    out_ref[...] = ssum * dinv_ref[...] + b_ref[...]


def _tc2(accp, g, dinv, b):
    blk = 2000
    return pl.pallas_call(
        _tc2_body,
        grid=(N_NODES // blk,),
        in_specs=[
            pl.BlockSpec((2, blk, NHID), lambda i: (0, i, 0)),
            pl.BlockSpec((blk, NHID), lambda i: (i, 0)),
            pl.BlockSpec((blk, 1), lambda i: (i, 0)),
            pl.BlockSpec((1, NHID), lambda i: (0, 0)),
        ],
        out_specs=pl.BlockSpec((blk, NHID), lambda i: (i, 0)),
        out_shape=jax.ShapeDtypeStruct((N_NODES, NHID), jnp.float32),
    )(accp, g, dinv, b)


@jax.jit
def kernel(x, edge_index, W, b):
    ei = edge_index.astype(jnp.int32)
    degp, dst2 = _deg_kernel(ei)
    g, dinv = _scale(degp, x, W)
    zc = jnp.zeros((CHUNK, NHID), jnp.float32)
    accp = _edge_kernel(ei, dst2, g, zc)
    return _tc2(accp, g, dinv, b.reshape(1, NHID))


# speedup vs baseline: 1.0389x; 1.0015x over previous
import functools

import jax
import jax.numpy as jnp
from jax import lax
from jax.experimental import pallas as pl
from jax.experimental.pallas import tpu as pltpu
from jax.experimental.pallas import tpu_sc as plsc

N_NODES = 10000
NFEAT = 128
NHID = 128

NC = 2
NS = 16
NW = NC * NS
L = 16

N_PAD = 10240
ROWS_PER_TILE = N_PAD // NS

N_EDGES = 320000
CHUNK = 128
NCHUNK_TOT = N_EDGES // CHUNK
NCHUNK = 80
H = NCHUNK // 2
VPC = CHUNK // L
PART = NCHUNK_TOT - (NW - 1) * NCHUNK

_mesh = plsc.VectorSubcoreMesh(core_axis_name="c", subcore_axis_name="s")


def _nh(wid, h):
    n_real = jnp.clip(NCHUNK_TOT - wid * NCHUNK, 0, NCHUNK)
    return jnp.clip(n_real - h * H, 0, H)


@functools.partial(
    pl.kernel,
    out_type=(
        jax.ShapeDtypeStruct((NW, N_PAD), jnp.float32),
        jax.ShapeDtypeStruct((NW * NCHUNK, CHUNK), jnp.int32),
    ),
    mesh=_mesh,
    scratch_types=[
        pltpu.VMEM((H * CHUNK,), jnp.int32),
        pltpu.VMEM((H, CHUNK), jnp.int32),
        pltpu.VMEM((N_PAD,), jnp.float32),
    ],
    compiler_params=pltpu.CompilerParams(needs_layout_passes=False),
)
def _deg_kernel(ei_hbm, out_hbm, dst2_hbm, idxf_v, idx2_v, deg_v):
    c = lax.axis_index("c")
    s = lax.axis_index("s")
    wid = c * NS + s
    zeros = jnp.zeros((L,), jnp.float32)
    ones = jnp.ones((L,), jnp.float32)

    def zero_body(i, _):
        for u in range(4):
            deg_v[pl.ds((4 * i + u) * L, L)] = zeros
        return 0

    lax.fori_loop(0, N_PAD // L // 4, zero_body, 0)

    for h in range(2):
        nh = _nh(wid, h)
        e0 = wid * NCHUNK * CHUNK + h * H * CHUNK

        @pl.when(nh >= H)
        def _full():
            pltpu.sync_copy(ei_hbm.at[1, pl.ds(e0, H * CHUNK)], idxf_v)

        @pl.when(jnp.logical_and(nh > 0, nh < H))
        def _tail():
            pltpu.sync_copy(ei_hbm.at[1, pl.ds(e0, PART * CHUNK)],
                            idxf_v.at[pl.ds(0, PART * CHUNK)])

        @pl.when(nh > 0)
        def _half():
            def body(i4, _):
                for u in range(4):
                    i = i4 * 4 + u
                    idx = idxf_v[pl.ds(i * L, L)]
                    plsc.addupdate_scatter(deg_v, [idx], ones)
                    idx2_v[i // VPC, pl.ds((i % VPC) * L, L)] = idx
                return 0

            lax.fori_loop(0, nh * VPC // 4, body, 0)
            pltpu.sync_copy(idx2_v,
                            dst2_hbm.at[pl.ds(wid * NCHUNK + h * H, H)])

    pltpu.sync_copy(deg_v, out_hbm.at[wid])


def _scale_body(degp_ref, x_ref, w_ref, g_ref, dinv_ref):
    deg = jnp.sum(degp_ref[...], axis=0) + 1.0
    dinv = lax.rsqrt(deg)[:N_NODES]
    h = jnp.dot(x_ref[...], w_ref[...], preferred_element_type=jnp.float32)
    g_ref[...] = h * dinv[:, None]
    dinv_ref[...] = dinv[:, None]


def _scale(degp, x, w):
    return pl.pallas_call(
        _scale_body,
        out_shape=(
            jax.ShapeDtypeStruct((N_NODES, NHID), jnp.float32),
            jax.ShapeDtypeStruct((N_NODES, 1), jnp.float32),
        ),
    )(degp, x, w)


@functools.partial(
    pl.kernel,
    out_type=jax.ShapeDtypeStruct((NC, N_PAD, NHID), jnp.float32),
    mesh=_mesh,
    scratch_types=[
        pltpu.VMEM((H * CHUNK,), jnp.int32),
        pltpu.VMEM((H, CHUNK), jnp.int32),
        pltpu.VMEM((2, CHUNK, NHID), jnp.float32),
        pltpu.VMEM_SHARED((N_PAD, NHID), jnp.float32),
        pltpu.SemaphoreType.DMA,
        pltpu.SemaphoreType.DMA,
    ],
)
def _edge_kernel(ei_hbm, dst2_hbm, g_hbm, zeros_hbm, out_hbm,
                 src_v, dst_v, rows_v, acc_sh, sem, sem_s):
    c = lax.axis_index("c")
    s = lax.axis_index("s")
    wid = c * NS + s

    pltpu.sync_copy(zeros_hbm, rows_v.at[0])
    base_row = s * ROWS_PER_TILE
    NZ = ROWS_PER_TILE // CHUNK
    for k in range(NZ):
        pltpu.async_copy(rows_v.at[0],
                         acc_sh.at[pl.ds(base_row + k * CHUNK, CHUNK)],
                         sem_s)

    nh0 = _nh(wid, 0)
    e00 = wid * NCHUNK * CHUNK

    @pl.when(nh0 >= H)
    def _full0():
        pltpu.sync_copy(ei_hbm.at[0, pl.ds(e00, H * CHUNK)], src_v)

    @pl.when(jnp.logical_and(nh0 > 0, nh0 < H))
    def _tail0():
        pltpu.sync_copy(ei_hbm.at[0, pl.ds(e00, PART * CHUNK)],
                        src_v.at[pl.ds(0, PART * CHUNK)])

    @pl.when(nh0 > 0)
    def _dst0():
        pltpu.sync_copy(dst2_hbm.at[pl.ds(wid * NCHUNK, H)], dst_v)

    for k in range(NZ):
        pltpu.make_async_copy(
            rows_v.at[0], acc_sh.at[pl.ds(base_row + k * CHUNK, CHUNK)],
            sem_s).wait()
    plsc.subcore_barrier()

    for h in range(2):
        nh = _nh(wid, h)
        e0 = wid * NCHUNK * CHUNK + h * H * CHUNK

        if h == 1:
            @pl.when(nh >= H)
            def _full():
                pltpu.sync_copy(ei_hbm.at[0, pl.ds(e0, H * CHUNK)], src_v)

            @pl.when(jnp.logical_and(nh > 0, nh < H))
            def _tail():
                pltpu.sync_copy(ei_hbm.at[0, pl.ds(e0, PART * CHUNK)],
                                src_v.at[pl.ds(0, PART * CHUNK)])

            @pl.when(nh > 0)
            def _dsth():
                pltpu.sync_copy(
                    dst2_hbm.at[pl.ds(wid * NCHUNK + h * H, H)], dst_v)

        def _g(j):
            pltpu.async_copy(g_hbm.at[src_v.at[pl.ds(j * CHUNK, CHUNK)]],
                             rows_v.at[j & 1], sem)

        def _wait_g(j):
            pltpu.make_async_copy(
                g_hbm.at[src_v.at[pl.ds(j * CHUNK, CHUNK)]],
                rows_v.at[j & 1], sem).wait()

        def _s(j):
            pltpu.async_copy(rows_v.at[j & 1], acc_sh.at[dst_v.at[j]],
                             sem_s, add=True)

        def _wait_s(j):
            pltpu.make_async_copy(rows_v.at[j & 1], acc_sh.at[dst_v.at[j]],
                                  sem_s).wait()

        @pl.when(nh > 0)
        def _half():
            _g(0)

            @pl.when(nh > 1)
            def _p2():
                _g(1)

            _wait_g(0)
            _s(0)

            @pl.when(nh > 1)
            def _steady():
                def chunk_body(j, _):
                    _wait_s(j - 1)
                    _g(j + 1)
                    _wait_g(j)
                    _s(j)
                    return 0

                lax.fori_loop(1, nh - 1, chunk_body, 0)
                _wait_s(nh - 2)
                _wait_g(nh - 1)
                _s(nh - 1)
                _wait_s(nh - 1)

            @pl.when(nh == 1)
            def _one():
                _wait_s(0)

    plsc.subcore_barrier()
    pltpu.sync_copy(acc_sh.at[pl.ds(base_row, ROWS_PER_TILE)],
                    out_hbm.at[c, pl.ds(base_row, ROWS_PER_TILE)])


def _tc2_body(accp_ref, g_ref, dinv_ref, b_ref, out_ref):
    ssum = accp_ref[0] + accp_ref[1] + g_ref[...]
    out_ref[...] = ssum * dinv_ref[...] + b_ref[...]


def _tc2(accp, g, dinv, b):
    blk = 2000
    return pl.pallas_call(
        _tc2_body,
        grid=(N_NODES // blk,),
        in_specs=[
            pl.BlockSpec((2, blk, NHID), lambda i: (0, i, 0)),
            pl.BlockSpec((blk, NHID), lambda i: (i, 0)),
            pl.BlockSpec((blk, 1), lambda i: (i, 0)),
            pl.BlockSpec((1, NHID), lambda i: (0, 0)),
        ],
        out_specs=pl.BlockSpec((blk, NHID), lambda i: (i, 0)),
        out_shape=jax.ShapeDtypeStruct((N_NODES, NHID), jnp.float32),
    )(accp, g, dinv, b)


@jax.jit
def kernel(x, edge_index, W, b):
    ei = edge_index.astype(jnp.int32)
    degp, dst2 = _deg_kernel(ei)
    g, dinv = _scale(degp, x, W)
    zc = jnp.zeros((CHUNK, NHID), jnp.float32)
    accp = _edge_kernel(ei, dst2, g, zc)
    return _tc2(accp, g, dinv, b.reshape(1, NHID))
